# Initial kernel scaffold; baseline (speedup 1.0000x reference)
import jax
import jax.numpy as jnp
from jax.experimental import pallas as pl


def kernel(feat, edge_index, W2, b2, W3, b3):
    raise NotImplementedError("write your pallas kernel here")


# speedup vs baseline: 6.8918x; 6.8918x over previous
import functools

import jax
import jax.numpy as jnp
from jax import lax
from jax.experimental import pallas as pl
from jax.experimental.pallas import tpu as pltpu
from jax.experimental.pallas import tpu_sc as plsc

_N = 10000
_E = 320000
_F = 128
_NHID = 256
_NCLASS = 64
_EPS = 0.03

_NC = 2
_NS = 16
_NW = _NC * _NS
_E_PER_W = _E // _NW
_CHUNK = 80
_NCH = _E_PER_W // _CHUNK
_N_PAD = 10240
_ROWS_PER_TILE = _N_PAD // _NS
_ZR = 160


def _sc_agg_body(feat_hbm, src_hbm, dst_hbm, out_hbm,
                 src_v, dst_v, rows_v, agg_sh, sem):
    c = lax.axis_index("c")
    s = lax.axis_index("s")
    wid = c * _NS + s

    zv = jnp.zeros((16,), jnp.float32)

    def zrow(i, carry):
        for k in range(_F // 16):
            rows_v[i, pl.ds(k * 16, 16)] = zv
        return carry

    lax.fori_loop(0, _CHUNK, zrow, 0)
    base_rows = s * _ROWS_PER_TILE
    for k in range(_ROWS_PER_TILE // _CHUNK):
        pltpu.sync_copy(rows_v, agg_sh.at[pl.ds(base_rows + k * _CHUNK, _CHUNK)])
    plsc.subcore_barrier()

    pltpu.sync_copy(src_hbm.at[wid], src_v)
    pltpu.sync_copy(dst_hbm.at[wid], dst_v)

    def body(j, carry):
        pltpu.async_copy(feat_hbm.at[src_v.at[j]], rows_v, sem).wait()
        pltpu.sync_copy(rows_v, agg_sh.at[dst_v.at[j]], add=True)
        return carry

    lax.fori_loop(0, _NCH, body, 0)
    plsc.subcore_barrier()

    sl = pl.ds(base_rows, _ROWS_PER_TILE)
    pltpu.sync_copy(agg_sh.at[sl], out_hbm.at[c, sl])


_sc_agg = functools.partial(
    pl.kernel,
    out_type=jax.ShapeDtypeStruct((_NC, _N_PAD, _F), jnp.float32),
    mesh=plsc.VectorSubcoreMesh(core_axis_name="c", subcore_axis_name="s"),
    scratch_types=[
        pltpu.VMEM((_NCH, _CHUNK), jnp.int32),
        pltpu.VMEM((_NCH, _CHUNK), jnp.int32),
        pltpu.VMEM((_CHUNK, _F), jnp.float32),
        pltpu.VMEM_SHARED((_N_PAD, _F), jnp.float32),
        pltpu.SemaphoreType.DMA,
    ],
)(_sc_agg_body)


_MB = 1000


def _mlp_body(feat_ref, agg_ref, w2_ref, b2_ref, w3_ref, b3_ref, out_ref):
    h = (1.0 + _EPS) * feat_ref[...] + agg_ref[0] + agg_ref[1]
    x = jnp.dot(h, w2_ref[...], preferred_element_type=jnp.float32,
                precision=lax.Precision.HIGHEST)
    x = jnp.maximum(x + b2_ref[...], 0.0)
    y = jnp.dot(x, w3_ref[...], preferred_element_type=jnp.float32,
                precision=lax.Precision.HIGHEST)
    y = y + b3_ref[...]
    m = jnp.max(y, axis=1, keepdims=True)
    e = jnp.exp(y - m)
    out_ref[...] = (y - m) - jnp.log(jnp.sum(e, axis=1, keepdims=True))


def kernel(feat, edge_index, W2, b2, W3, b3):
    src = edge_index[0].reshape(_NW, _NCH, _CHUNK)
    dst = edge_index[1].reshape(_NW, _NCH, _CHUNK)
    agg = _sc_agg(feat, src, dst)

    grid = (_N // _MB,)
    out = pl.pallas_call(
        _mlp_body,
        grid=grid,
        in_specs=[
            pl.BlockSpec((_MB, _F), lambda i: (i, 0)),
            pl.BlockSpec((_NC, _MB, _F), lambda i: (0, i, 0)),
            pl.BlockSpec((_F, _NHID), lambda i: (0, 0)),
            pl.BlockSpec((1, _NHID), lambda i: (0, 0)),
            pl.BlockSpec((_NHID, _NCLASS), lambda i: (0, 0)),
            pl.BlockSpec((1, _NCLASS), lambda i: (0, 0)),
        ],
        out_specs=pl.BlockSpec((_MB, _NCLASS), lambda i: (i, 0)),
        out_shape=jax.ShapeDtypeStruct((_N, _NCLASS), jnp.float32),
    )(feat, agg, W2, b2.reshape(1, _NHID), W3, b3.reshape(1, _NCLASS))
    return out
